# baseline (speedup 1.0000x reference)
import functools

import jax
import jax.numpy as jnp
from jax.experimental import pallas as pl
from jax.experimental.pallas import tpu as pltpu

B, S, D = 1, 2048, 768
E, K, DFF = 8, 2, 1024
T = B * S
TB = 2048
NTB = T // TB
NDC = 2
DC = DFF // NDC


def _router_kernel(x_ref, wr_ref, w_ref):
    x = x_ref[...]
    lane = jax.lax.broadcasted_iota(jnp.int32, (TB, E), 1)
    l = jnp.dot(x, wr_ref[...], preferred_element_type=jnp.float32)
    l = jnp.where(lane < E - 1, l, -1e30)
    m1 = jnp.max(l, axis=1, keepdims=True)
    idx1 = jnp.min(jnp.where(l == m1, lane, E + 9), axis=1, keepdims=True)
    l2 = jnp.where(lane == idx1, -1e30, l)
    m2 = jnp.max(l2, axis=1, keepdims=True)
    idx2 = jnp.min(jnp.where(l2 == m2, lane, E + 9), axis=1, keepdims=True)
    p2 = jnp.exp(m2 - m1)
    denom = 1.0 + p2
    wfull = jnp.where(lane == idx1 + 1, 1.0 / denom, 0.0)
    wfull = wfull + jnp.where(lane == idx2 + 1, p2 / denom, 0.0)
    wfull = wfull + jnp.where(lane == 0, 1.0, 0.0)
    w_ref[...] = wfull


def _moe_dense_kernel(x_ref, w1_ref, w2_ref, w_ref, out_ref):
    e = pl.program_id(0)
    dc = pl.program_id(1)

    x = x_ref[...]
    h = jnp.dot(x, w1_ref[0], preferred_element_type=jnp.float32)
    h = h * jax.lax.logistic(h)

    onehot = (jax.lax.broadcasted_iota(jnp.int32, (E, 1), 0) == e).astype(
        jnp.float32)
    wcol = jax.lax.dot(w_ref[...], onehot,
                       precision=jax.lax.Precision.HIGHEST,
                       preferred_element_type=jnp.float32)
    h = (h * wcol).astype(jnp.bfloat16)
    y = jnp.dot(h, w2_ref[0], preferred_element_type=jnp.float32)

    @pl.when((e == 0) & (dc == 0))
    def _init():
        out_ref[...] = y

    @pl.when((e > 0) | (dc > 0))
    def _acc():
        out_ref[...] += y


def kernel(hidden_states, Wr, W1, W2, interpret=False):
    x = hidden_states.reshape(T, D)
    wr_pad = jnp.zeros((D, E), jnp.float32).at[:, : E - 1].set(Wr)
    x16 = x.astype(jnp.bfloat16)
    w1b = W1.astype(jnp.bfloat16)
    w2b = W2.astype(jnp.bfloat16)

    wfull = pl.pallas_call(
        _router_kernel,
        grid=(NTB,),
        in_specs=[
            pl.BlockSpec((TB, D), lambda tb: (tb, 0)),
            pl.BlockSpec((D, E), lambda tb: (0, 0)),
        ],
        out_specs=pl.BlockSpec((TB, E), lambda tb: (tb, 0)),
        out_shape=jax.ShapeDtypeStruct((T, E), jnp.float32),
        interpret=interpret,
    )(x, wr_pad)

    out = pl.pallas_call(
        _moe_dense_kernel,
        grid=(E, NDC),
        in_specs=[
            pl.BlockSpec((T, D), lambda e, dc: (0, 0)),
            pl.BlockSpec((1, D, DC), lambda e, dc: (e, 0, dc)),
            pl.BlockSpec((1, DC, D), lambda e, dc: (e, dc, 0)),
            pl.BlockSpec((T, E), lambda e, dc: (0, 0)),
        ],
        out_specs=pl.BlockSpec((T, D), lambda e, dc: (0, 0)),
        out_shape=jax.ShapeDtypeStruct((T, D), jnp.float32),
        interpret=interpret,
    )(x16, w1b, w2b, wfull)
    return out.reshape(B, S, D)


# speedup vs baseline: 1.0756x; 1.0756x over previous
import functools

import jax
import jax.numpy as jnp
from jax.experimental import pallas as pl
from jax.experimental.pallas import tpu as pltpu

B, S, D = 1, 2048, 768
E, K, DFF = 8, 2, 1024
T = B * S
TB = 2048
NTB = T // TB
NDC = 1
DC = DFF // NDC


def _router_kernel(x_ref, wr_ref, w_ref):
    x = x_ref[...]
    lane = jax.lax.broadcasted_iota(jnp.int32, (TB, E), 1)
    l = jnp.dot(x, wr_ref[...], preferred_element_type=jnp.float32)
    l = jnp.where(lane < E - 1, l, -1e30)
    m1 = jnp.max(l, axis=1, keepdims=True)
    idx1 = jnp.min(jnp.where(l == m1, lane, E + 9), axis=1, keepdims=True)
    l2 = jnp.where(lane == idx1, -1e30, l)
    m2 = jnp.max(l2, axis=1, keepdims=True)
    idx2 = jnp.min(jnp.where(l2 == m2, lane, E + 9), axis=1, keepdims=True)
    p2 = jnp.exp(m2 - m1)
    denom = 1.0 + p2
    wfull = jnp.where(lane == idx1 + 1, 1.0 / denom, 0.0)
    wfull = wfull + jnp.where(lane == idx2 + 1, p2 / denom, 0.0)
    wfull = wfull + jnp.where(lane == 0, 1.0, 0.0)
    w_ref[...] = wfull


def _moe_dense_kernel(x_ref, w1_ref, w2_ref, w_ref, out_ref):
    e = pl.program_id(0)
    dc = pl.program_id(1)

    x = x_ref[...]
    h = jnp.dot(x, w1_ref[0], preferred_element_type=jnp.float32)
    h = h * jax.lax.logistic(h)

    onehot = (jax.lax.broadcasted_iota(jnp.int32, (E, 1), 0) == e).astype(
        jnp.float32)
    wcol = jax.lax.dot(w_ref[...], onehot,
                       precision=jax.lax.Precision.HIGHEST,
                       preferred_element_type=jnp.float32)
    h = (h * wcol).astype(jnp.bfloat16)
    y = jnp.dot(h, w2_ref[0], preferred_element_type=jnp.float32)

    @pl.when((e == 0) & (dc == 0))
    def _init():
        out_ref[...] = y

    @pl.when((e > 0) | (dc > 0))
    def _acc():
        out_ref[...] += y


def kernel(hidden_states, Wr, W1, W2, interpret=False):
    x = hidden_states.reshape(T, D)
    wr_pad = jnp.zeros((D, E), jnp.float32).at[:, : E - 1].set(Wr)
    x16 = x.astype(jnp.bfloat16)
    w1b = W1.astype(jnp.bfloat16)
    w2b = W2.astype(jnp.bfloat16)

    wfull = pl.pallas_call(
        _router_kernel,
        grid=(NTB,),
        in_specs=[
            pl.BlockSpec((TB, D), lambda tb: (tb, 0)),
            pl.BlockSpec((D, E), lambda tb: (0, 0)),
        ],
        out_specs=pl.BlockSpec((TB, E), lambda tb: (tb, 0)),
        out_shape=jax.ShapeDtypeStruct((T, E), jnp.float32),
        interpret=interpret,
    )(x, wr_pad)

    out = pl.pallas_call(
        _moe_dense_kernel,
        grid=(E, NDC),
        in_specs=[
            pl.BlockSpec((T, D), lambda e, dc: (0, 0)),
            pl.BlockSpec((1, D, DC), lambda e, dc: (e, 0, dc)),
            pl.BlockSpec((1, DC, D), lambda e, dc: (e, dc, 0)),
            pl.BlockSpec((T, E), lambda e, dc: (0, 0)),
        ],
        out_specs=pl.BlockSpec((T, D), lambda e, dc: (0, 0)),
        out_shape=jax.ShapeDtypeStruct((T, D), jnp.float32),
        interpret=interpret,
    )(x16, w1b, w2b, wfull)
    return out.reshape(B, S, D)


# speedup vs baseline: 1.3579x; 1.2624x over previous
import functools

import jax
import jax.numpy as jnp
from jax.experimental import pallas as pl
from jax.experimental.pallas import tpu as pltpu

B, S, D = 1, 2048, 768
E, K, DFF = 8, 2, 1024
T = B * S
TB = 2048
NTB = T // TB
NDC = 1
DC = DFF // NDC


def _router_kernel(x_ref, wr_ref, w_ref):
    x = x_ref[...]
    lane = jax.lax.broadcasted_iota(jnp.int32, (TB, E), 1)
    l = jnp.dot(x, wr_ref[...], preferred_element_type=jnp.float32)
    l = jnp.where(lane < E - 1, l, -1e30)
    m1 = jnp.max(l, axis=1, keepdims=True)
    idx1 = jnp.min(jnp.where(l == m1, lane, E + 9), axis=1, keepdims=True)
    l2 = jnp.where(lane == idx1, -1e30, l)
    m2 = jnp.max(l2, axis=1, keepdims=True)
    idx2 = jnp.min(jnp.where(l2 == m2, lane, E + 9), axis=1, keepdims=True)
    p2 = jnp.exp(m2 - m1)
    denom = 1.0 + p2
    wfull = jnp.where(lane == idx1 + 1, 1.0 / denom, 0.0)
    wfull = wfull + jnp.where(lane == idx2 + 1, p2 / denom, 0.0)
    wfull = wfull + jnp.where(lane == 0, 1.0, 0.0)
    w_ref[...] = wfull


def _moe_dense_kernel(x_ref, w1_ref, w2_ref, w_ref, out_ref):
    e = pl.program_id(0)
    dc = pl.program_id(1)

    x = x_ref[...]
    h = jnp.dot(x, w1_ref[0], preferred_element_type=jnp.float32)
    h = h * jax.lax.logistic(h)
    y = jnp.dot(h.astype(jnp.bfloat16), w2_ref[0],
                preferred_element_type=jnp.float32)

    lane = jax.lax.broadcasted_iota(jnp.int32, (TB, E), 1)
    wcol = jnp.sum(jnp.where(lane == e, w_ref[...], 0.0), axis=1, keepdims=True)
    contrib = y * wcol

    @pl.when(e == 0)
    def _init():
        out_ref[...] = contrib

    @pl.when(e > 0)
    def _acc():
        out_ref[...] += contrib


def kernel(hidden_states, Wr, W1, W2, interpret=False):
    x = hidden_states.reshape(T, D)
    wr_pad = jnp.zeros((D, E), jnp.float32).at[:, : E - 1].set(Wr)
    x16 = x.astype(jnp.bfloat16)
    w1b = W1.astype(jnp.bfloat16)
    w2b = W2.astype(jnp.bfloat16)

    wfull = pl.pallas_call(
        _router_kernel,
        grid=(NTB,),
        in_specs=[
            pl.BlockSpec((TB, D), lambda tb: (tb, 0)),
            pl.BlockSpec((D, E), lambda tb: (0, 0)),
        ],
        out_specs=pl.BlockSpec((TB, E), lambda tb: (tb, 0)),
        out_shape=jax.ShapeDtypeStruct((T, E), jnp.float32),
        interpret=interpret,
    )(x, wr_pad)

    out = pl.pallas_call(
        _moe_dense_kernel,
        grid=(E, NDC),
        in_specs=[
            pl.BlockSpec((T, D), lambda e, dc: (0, 0)),
            pl.BlockSpec((1, D, DC), lambda e, dc: (e, 0, dc)),
            pl.BlockSpec((1, DC, D), lambda e, dc: (e, dc, 0)),
            pl.BlockSpec((T, E), lambda e, dc: (0, 0)),
        ],
        out_specs=pl.BlockSpec((T, D), lambda e, dc: (0, 0)),
        out_shape=jax.ShapeDtypeStruct((T, D), jnp.float32),
        interpret=interpret,
    )(x16, w1b, w2b, wfull)
    return out.reshape(B, S, D)
